# Initial kernel scaffold; baseline (speedup 1.0000x reference)
import jax
import jax.numpy as jnp
from jax.experimental import pallas as pl


def kernel(x, w_gate, Wq, kv_w, kv_b, W_out, task_bh):
    raise NotImplementedError("write your pallas kernel here")


# speedup vs baseline: 1.9027x; 1.9027x over previous
import jax
import jax.numpy as jnp
from jax.experimental import pallas as pl

DIM = 768
E = 16
K = 8
HD = 96
B = 4
N = 2048
T = B * N
BT = 256
BQ = 256
NBT = T // BT
NQ = N // BQ


def _route_proj_kernel(x_ref, wg_ref, wq_ref, kvw_ref, kvb_ref,
                       q_ref, k_ref, v_ref, gs_ref, is_ref,
                       fr_ref, ps_ref, zs_ref):
    x = x_ref[...]
    kv = jnp.dot(x, kvw_ref[...], preferred_element_type=jnp.float32)
    kv = kv + kvb_ref[...]
    k_ref[...] = kv[:, :HD]
    v_ref[...] = kv[:, HD:]
    logits = jnp.dot(x, wg_ref[...], preferred_element_type=jnp.float32)
    m = jnp.max(logits, axis=-1, keepdims=True)
    ex = jnp.exp(logits - m)
    se = jnp.sum(ex, axis=-1, keepdims=True)
    p = ex / se
    lse = m + jnp.log(se)
    zs_ref[...] = jnp.broadcast_to(jnp.sum(lse * lse), (1, 1, 8))
    eidx = jax.lax.broadcasted_iota(jnp.int32, (BT, E), 1)
    rank = jnp.zeros((BT, E), jnp.int32)
    for j in range(E):
        pj = p[:, j:j + 1]
        rank = rank + jnp.where((pj > p) | ((pj == p) & (j < eidx)), 1, 0)
    sel = rank < K
    self32 = sel.astype(jnp.float32)
    gm = self32 * p
    g = gm / (jnp.sum(gm, axis=-1, keepdims=True) + 1e-6)
    slot = jnp.zeros((BT, E), jnp.int32)
    for j in range(E):
        sj = jnp.where(sel[:, j:j + 1], 1, 0)
        slot = slot + jnp.where(eidx > j, sj, 0)
    y = jnp.dot(x, wq_ref[...], preferred_element_type=jnp.float32)
    eidxf = eidx.astype(jnp.float32)
    gcols = []
    icols = []
    for kk in range(K):
        sk = self32 * (slot == kk).astype(jnp.float32)
        acc = jnp.zeros((BT, HD), jnp.float32)
        for e in range(E):
            acc = acc + sk[:, e:e + 1] * y[:, e * HD:(e + 1) * HD]
        q_ref[kk, :, :] = acc
        gcols.append(jnp.sum(sk * g, axis=-1, keepdims=True))
        icols.append(jnp.sum(sk * eidxf, axis=-1, keepdims=True))
    gs_ref[0] = jnp.concatenate(gcols, axis=-1)
    is_ref[0] = jnp.concatenate(icols, axis=-1)
    fr_ref[0] = jnp.sum(self32, axis=0, keepdims=True)
    ps_ref[0] = jnp.sum(p, axis=0, keepdims=True)


def _attn_kernel(q_ref, k_ref, v_ref, o_ref):
    q = q_ref[0]
    kk = k_ref[0]
    vv = v_ref[0]
    s = jax.lax.dot_general(q, kk, (((1,), (1,)), ((), ())),
                            preferred_element_type=jnp.float32)
    s = s * (HD ** -0.5)
    m = jnp.max(s, axis=-1, keepdims=True)
    e = jnp.exp(s - m)
    denom = jnp.sum(e, axis=-1, keepdims=True)
    o = jnp.dot(e, vv, preferred_element_type=jnp.float32)
    o_ref[0] = o / denom


def _reduce_kernel(o_ref, gs_ref, is_ref, wo_ref, out_ref):
    gs = gs_ref[0]
    idx = is_ref[0]
    zcols = []
    for e in range(E):
        acc = jnp.zeros((BT, HD), jnp.float32)
        for kk in range(K):
            w = jnp.where(idx[:, kk:kk + 1] == float(e),
                          gs[:, kk:kk + 1], 0.0)
            acc = acc + w * o_ref[kk]
        zcols.append(acc)
    z = jnp.concatenate(zcols, axis=-1)
    out_ref[...] = jnp.dot(z, wo_ref[...], preferred_element_type=jnp.float32)


def kernel(x, w_gate, Wq, kv_w, kv_b, W_out, task_bh):
    xf = x.reshape(T, DIM)
    wg = w_gate[task_bh]
    wq_flat = jnp.transpose(Wq, (1, 0, 2)).reshape(DIM, E * HD)
    wo_flat = W_out.reshape(E * HD, DIM)
    kvb2 = kv_b.reshape(1, 2 * HD)

    q, k_, v_, gs, is_, fr, ps, zs = pl.pallas_call(
        _route_proj_kernel,
        grid=(NBT,),
        in_specs=[
            pl.BlockSpec((BT, DIM), lambda i: (i, 0)),
            pl.BlockSpec((DIM, E), lambda i: (0, 0)),
            pl.BlockSpec((DIM, E * HD), lambda i: (0, 0)),
            pl.BlockSpec((DIM, 2 * HD), lambda i: (0, 0)),
            pl.BlockSpec((1, 2 * HD), lambda i: (0, 0)),
        ],
        out_specs=[
            pl.BlockSpec((K, BT, HD), lambda i: (0, i, 0)),
            pl.BlockSpec((BT, HD), lambda i: (i, 0)),
            pl.BlockSpec((BT, HD), lambda i: (i, 0)),
            pl.BlockSpec((1, BT, K), lambda i: (i, 0, 0)),
            pl.BlockSpec((1, BT, K), lambda i: (i, 0, 0)),
            pl.BlockSpec((1, 1, E), lambda i: (i, 0, 0)),
            pl.BlockSpec((1, 1, E), lambda i: (i, 0, 0)),
            pl.BlockSpec((1, 1, 8), lambda i: (i, 0, 0)),
        ],
        out_shape=[
            jax.ShapeDtypeStruct((K, T, HD), jnp.float32),
            jax.ShapeDtypeStruct((T, HD), jnp.float32),
            jax.ShapeDtypeStruct((T, HD), jnp.float32),
            jax.ShapeDtypeStruct((NBT, BT, K), jnp.float32),
            jax.ShapeDtypeStruct((NBT, BT, K), jnp.float32),
            jax.ShapeDtypeStruct((NBT, 1, E), jnp.float32),
            jax.ShapeDtypeStruct((NBT, 1, E), jnp.float32),
            jax.ShapeDtypeStruct((NBT, 1, 8), jnp.float32),
        ],
    )(xf, wg, wq_flat, kv_w, kvb2)

    o = pl.pallas_call(
        _attn_kernel,
        grid=(B, K, NQ),
        in_specs=[
            pl.BlockSpec((1, BQ, HD), lambda b, h, i: (h, b * NQ + i, 0)),
            pl.BlockSpec((1, N, HD), lambda b, h, i: (0, b, 0)),
            pl.BlockSpec((1, N, HD), lambda b, h, i: (0, b, 0)),
        ],
        out_specs=pl.BlockSpec((1, BQ, HD), lambda b, h, i: (h, b * NQ + i, 0)),
        out_shape=jax.ShapeDtypeStruct((K, T, HD), jnp.float32),
    )(q, k_.reshape(1, T, HD), v_.reshape(1, T, HD))

    out2d = pl.pallas_call(
        _reduce_kernel,
        grid=(NBT,),
        in_specs=[
            pl.BlockSpec((K, BT, HD), lambda i: (0, i, 0)),
            pl.BlockSpec((1, BT, K), lambda i: (i, 0, 0)),
            pl.BlockSpec((1, BT, K), lambda i: (i, 0, 0)),
            pl.BlockSpec((E * HD, DIM), lambda i: (0, 0)),
        ],
        out_specs=pl.BlockSpec((BT, DIM), lambda i: (i, 0)),
        out_shape=jax.ShapeDtypeStruct((T, DIM), jnp.float32),
    )(o, gs, is_, wo_flat)

    out = out2d.reshape(B, N, DIM)

    zsum = jnp.sum(zs[:, 0, 0])
    zloss = 0.001 * zsum / T
    freqs = jnp.sum(fr[:, 0, :], axis=0)
    psum = jnp.sum(ps[:, 0, :], axis=0)
    freqs_n = freqs / (jnp.sum(freqs) + 1e-9)
    pm = psum / T
    pm_n = pm / (jnp.sum(pm) + 1e-9)
    switchloss = 0.1 * E * jnp.sum(pm_n * freqs_n)
    aux_loss = zloss + switchloss
    return out, aux_loss


# speedup vs baseline: 1.9565x; 1.0283x over previous
import jax
import jax.numpy as jnp
from jax.experimental import pallas as pl

DIM = 768
E = 16
K = 8
HD = 96
B = 4
N = 2048
T = B * N
BT = 256
BQ = 512
NBT = T // BT
NQ = N // BQ


def _route_proj_kernel(x_ref, wg_ref, wq_ref, kvw_ref, kvb_ref,
                       q_ref, k_ref, v_ref, gs_ref, is_ref,
                       fr_ref, ps_ref, zs_ref):
    x = x_ref[...]
    kv = jnp.dot(x, kvw_ref[...], preferred_element_type=jnp.float32)
    kv = kv + kvb_ref[...]
    k_ref[...] = kv[:, :HD]
    v_ref[...] = kv[:, HD:]
    logits = jnp.dot(x, wg_ref[...], preferred_element_type=jnp.float32)
    m = jnp.max(logits, axis=-1, keepdims=True)
    ex = jnp.exp(logits - m)
    se = jnp.sum(ex, axis=-1, keepdims=True)
    p = ex / se
    lse = m + jnp.log(se)
    zs_ref[...] = jnp.broadcast_to(jnp.sum(lse * lse), (1, 1, 8))
    eidx = jax.lax.broadcasted_iota(jnp.int32, (BT, E), 1)
    rank = jnp.zeros((BT, E), jnp.int32)
    for j in range(E):
        pj = p[:, j:j + 1]
        rank = rank + jnp.where((pj > p) | ((pj == p) & (j < eidx)), 1, 0)
    sel = rank < K
    self32 = sel.astype(jnp.float32)
    gm = self32 * p
    g = gm / (jnp.sum(gm, axis=-1, keepdims=True) + 1e-6)
    slot = jnp.zeros((BT, E), jnp.int32)
    for j in range(E):
        sj = jnp.where(sel[:, j:j + 1], 1, 0)
        slot = slot + jnp.where(eidx > j, sj, 0)
    y = jnp.dot(x, wq_ref[...], preferred_element_type=jnp.float32)
    eidxf = eidx.astype(jnp.float32)
    gcols = []
    icols = []
    for kk in range(K):
        sk = self32 * (slot == kk).astype(jnp.float32)
        acc = jnp.zeros((BT, HD), jnp.float32)
        for e in range(E):
            acc = acc + sk[:, e:e + 1] * y[:, e * HD:(e + 1) * HD]
        q_ref[kk, :, :] = acc
        gcols.append(jnp.sum(sk * g, axis=-1, keepdims=True))
        icols.append(jnp.sum(sk * eidxf, axis=-1, keepdims=True))
    gs_ref[0] = jnp.concatenate(gcols, axis=-1)
    is_ref[0] = jnp.concatenate(icols, axis=-1)
    fr_ref[0] = jnp.sum(self32, axis=0, keepdims=True)
    ps_ref[0] = jnp.sum(p, axis=0, keepdims=True)


def _attn_kernel(q_ref, k_ref, v_ref, o_ref):
    q = q_ref[0]
    kk = k_ref[0]
    vv = v_ref[0]
    s = jax.lax.dot_general(q, kk, (((1,), (1,)), ((), ())),
                            preferred_element_type=jnp.float32)
    s = s * (HD ** -0.5)
    m = jnp.max(s, axis=-1, keepdims=True)
    e = jnp.exp(s - m)
    denom = jnp.sum(e, axis=-1, keepdims=True)
    o = jnp.dot(e, vv, preferred_element_type=jnp.float32)
    o_ref[0] = o / denom


def _reduce_kernel(o_ref, gs_ref, is_ref, wo_ref, out_ref):
    gs = gs_ref[0]
    idx = is_ref[0]
    zcols = []
    for e in range(E):
        acc = jnp.zeros((BT, HD), jnp.float32)
        for kk in range(K):
            w = jnp.where(idx[:, kk:kk + 1] == float(e),
                          gs[:, kk:kk + 1], 0.0)
            acc = acc + w * o_ref[kk]
        zcols.append(acc)
    z = jnp.concatenate(zcols, axis=-1)
    out_ref[...] = jnp.dot(z, wo_ref[...], preferred_element_type=jnp.float32)


def kernel(x, w_gate, Wq, kv_w, kv_b, W_out, task_bh):
    xf = x.reshape(T, DIM)
    wg = w_gate[task_bh]
    wq_flat = jnp.transpose(Wq, (1, 0, 2)).reshape(DIM, E * HD)
    wo_flat = W_out.reshape(E * HD, DIM)
    kvb2 = kv_b.reshape(1, 2 * HD)

    q, k_, v_, gs, is_, fr, ps, zs = pl.pallas_call(
        _route_proj_kernel,
        grid=(NBT,),
        in_specs=[
            pl.BlockSpec((BT, DIM), lambda i: (i, 0)),
            pl.BlockSpec((DIM, E), lambda i: (0, 0)),
            pl.BlockSpec((DIM, E * HD), lambda i: (0, 0)),
            pl.BlockSpec((DIM, 2 * HD), lambda i: (0, 0)),
            pl.BlockSpec((1, 2 * HD), lambda i: (0, 0)),
        ],
        out_specs=[
            pl.BlockSpec((K, BT, HD), lambda i: (0, i, 0)),
            pl.BlockSpec((BT, HD), lambda i: (i, 0)),
            pl.BlockSpec((BT, HD), lambda i: (i, 0)),
            pl.BlockSpec((1, BT, K), lambda i: (i, 0, 0)),
            pl.BlockSpec((1, BT, K), lambda i: (i, 0, 0)),
            pl.BlockSpec((1, 1, E), lambda i: (i, 0, 0)),
            pl.BlockSpec((1, 1, E), lambda i: (i, 0, 0)),
            pl.BlockSpec((1, 1, 8), lambda i: (i, 0, 0)),
        ],
        out_shape=[
            jax.ShapeDtypeStruct((K, T, HD), jnp.float32),
            jax.ShapeDtypeStruct((T, HD), jnp.float32),
            jax.ShapeDtypeStruct((T, HD), jnp.float32),
            jax.ShapeDtypeStruct((NBT, BT, K), jnp.float32),
            jax.ShapeDtypeStruct((NBT, BT, K), jnp.float32),
            jax.ShapeDtypeStruct((NBT, 1, E), jnp.float32),
            jax.ShapeDtypeStruct((NBT, 1, E), jnp.float32),
            jax.ShapeDtypeStruct((NBT, 1, 8), jnp.float32),
        ],
    )(xf, wg, wq_flat, kv_w, kvb2)

    o = pl.pallas_call(
        _attn_kernel,
        grid=(B, K, NQ),
        in_specs=[
            pl.BlockSpec((1, BQ, HD), lambda b, h, i: (h, b * NQ + i, 0)),
            pl.BlockSpec((1, N, HD), lambda b, h, i: (0, b, 0)),
            pl.BlockSpec((1, N, HD), lambda b, h, i: (0, b, 0)),
        ],
        out_specs=pl.BlockSpec((1, BQ, HD), lambda b, h, i: (h, b * NQ + i, 0)),
        out_shape=jax.ShapeDtypeStruct((K, T, HD), jnp.float32),
    )(q, k_.reshape(1, T, HD), v_.reshape(1, T, HD))

    out2d = pl.pallas_call(
        _reduce_kernel,
        grid=(NBT,),
        in_specs=[
            pl.BlockSpec((K, BT, HD), lambda i: (0, i, 0)),
            pl.BlockSpec((1, BT, K), lambda i: (i, 0, 0)),
            pl.BlockSpec((1, BT, K), lambda i: (i, 0, 0)),
            pl.BlockSpec((E * HD, DIM), lambda i: (0, 0)),
        ],
        out_specs=pl.BlockSpec((BT, DIM), lambda i: (i, 0)),
        out_shape=jax.ShapeDtypeStruct((T, DIM), jnp.float32),
    )(o, gs, is_, wo_flat)

    out = out2d.reshape(B, N, DIM)

    zsum = jnp.sum(zs[:, 0, 0])
    zloss = 0.001 * zsum / T
    freqs = jnp.sum(fr[:, 0, :], axis=0)
    psum = jnp.sum(ps[:, 0, :], axis=0)
    freqs_n = freqs / (jnp.sum(freqs) + 1e-9)
    pm = psum / T
    pm_n = pm / (jnp.sum(pm) + 1e-9)
    switchloss = 0.1 * E * jnp.sum(pm_n * freqs_n)
    aux_loss = zloss + switchloss
    return out, aux_loss


# speedup vs baseline: 3.0892x; 1.5789x over previous
import jax
import jax.numpy as jnp
from jax.experimental import pallas as pl

DIM = 768
E = 16
K = 8
HD = 96
B = 4
N = 2048
T = B * N
BT = 512
BQ = 512
NBT = T // BT
NQ = N // BQ
EH = E * HD


def _route_proj_kernel(x_ref, wg_ref, wq_ref, kvw_ref, kvb_ref, sel_ref, r_ref,
                       q_ref, k_ref, v_ref, ws_ref, fr_ref, ps_ref, zs_ref):
    x = x_ref[...]
    kv = jnp.dot(x, kvw_ref[...], preferred_element_type=jnp.float32)
    kv = kv + kvb_ref[...]
    k_ref[...] = kv[:, :HD]
    v_ref[...] = kv[:, HD:]
    logits = jnp.dot(x, wg_ref[...], preferred_element_type=jnp.float32)
    m = jnp.max(logits, axis=-1, keepdims=True)
    ex = jnp.exp(logits - m)
    se = jnp.sum(ex, axis=-1, keepdims=True)
    p = ex / se
    lse = m + jnp.log(se)
    zs_ref[...] = jnp.broadcast_to(jnp.sum(lse * lse), (1, 1, 8))
    eidx = jax.lax.broadcasted_iota(jnp.int32, (BT, E), 1)
    rank = jnp.zeros((BT, E), jnp.int32)
    for j in range(E):
        pj = p[:, j:j + 1]
        rank = rank + jnp.where((pj > p) | ((pj == p) & (j < eidx)), 1, 0)
    sel = rank < K
    self32 = sel.astype(jnp.float32)
    gm = self32 * p
    g = gm / (jnp.sum(gm, axis=-1, keepdims=True) + 1e-6)
    slot = jnp.zeros((BT, E), jnp.int32)
    for j in range(E):
        sj = jnp.where(sel[:, j:j + 1], 1, 0)
        slot = slot + jnp.where(eidx > j, sj, 0)
    y = jnp.dot(x, wq_ref[...], preferred_element_type=jnp.float32)
    selmat = sel_ref[...]
    red = r_ref[...]
    wcols = []
    for kk in range(K):
        sk = self32 * (slot == kk).astype(jnp.float32)
        wide = jnp.dot(sk, selmat, preferred_element_type=jnp.float32)
        q_ref[kk, :, :] = jnp.dot(wide * y, red,
                                  preferred_element_type=jnp.float32)
        wcols.append(sk * g)
    ws_ref[...] = jnp.concatenate(wcols, axis=-1)
    fr_ref[0] = jnp.sum(self32, axis=0, keepdims=True)
    ps_ref[0] = jnp.sum(p, axis=0, keepdims=True)


def _attn_kernel(q_ref, k_ref, v_ref, o_ref):
    q = q_ref[0]
    kk = k_ref[0]
    vv = v_ref[0]
    s = jax.lax.dot_general(q, kk, (((1,), (1,)), ((), ())),
                            preferred_element_type=jnp.float32)
    s = s * (HD ** -0.5)
    m = jnp.max(s, axis=-1, keepdims=True)
    e = jnp.exp(s - m)
    denom = jnp.sum(e, axis=-1, keepdims=True)
    o = jnp.dot(e, vv, preferred_element_type=jnp.float32)
    o_ref[0] = o / denom


def _reduce_kernel(o_ref, ws_ref, sel_ref, tile_ref, wo_ref, out_ref):
    ws = ws_ref[...]
    selmat = sel_ref[...]
    tilem = tile_ref[...]
    z = jnp.zeros((BT, EH), jnp.float32)
    for kk in range(K):
        gk = ws[:, kk * E:(kk + 1) * E]
        wide = jnp.dot(gk, selmat, preferred_element_type=jnp.float32)
        rep = jnp.dot(o_ref[kk], tilem,
                      preferred_element_type=jnp.float32)
        z = z + wide * rep
    out_ref[...] = jnp.dot(z, wo_ref[...], preferred_element_type=jnp.float32)


def kernel(x, w_gate, Wq, kv_w, kv_b, W_out, task_bh):
    xf = x.reshape(T, DIM)
    wg = w_gate[task_bh]
    wq_flat = jnp.transpose(Wq, (1, 0, 2)).reshape(DIM, EH)
    wo_flat = W_out.reshape(EH, DIM)
    kvb2 = kv_b.reshape(1, 2 * HD)
    eye_e = jnp.eye(E, dtype=jnp.float32)
    selmat = jnp.repeat(eye_e, HD, axis=1).reshape(E, EH)
    red = jnp.tile(jnp.eye(HD, dtype=jnp.float32), (E, 1))
    tilem = jnp.tile(jnp.eye(HD, dtype=jnp.float32), (1, E))

    q, k_, v_, ws, fr, ps, zs = pl.pallas_call(
        _route_proj_kernel,
        grid=(NBT,),
        in_specs=[
            pl.BlockSpec((BT, DIM), lambda i: (i, 0)),
            pl.BlockSpec((DIM, E), lambda i: (0, 0)),
            pl.BlockSpec((DIM, EH), lambda i: (0, 0)),
            pl.BlockSpec((DIM, 2 * HD), lambda i: (0, 0)),
            pl.BlockSpec((1, 2 * HD), lambda i: (0, 0)),
            pl.BlockSpec((E, EH), lambda i: (0, 0)),
            pl.BlockSpec((EH, HD), lambda i: (0, 0)),
        ],
        out_specs=[
            pl.BlockSpec((K, BT, HD), lambda i: (0, i, 0)),
            pl.BlockSpec((BT, HD), lambda i: (i, 0)),
            pl.BlockSpec((BT, HD), lambda i: (i, 0)),
            pl.BlockSpec((BT, K * E), lambda i: (i, 0)),
            pl.BlockSpec((1, 1, E), lambda i: (i, 0, 0)),
            pl.BlockSpec((1, 1, E), lambda i: (i, 0, 0)),
            pl.BlockSpec((1, 1, 8), lambda i: (i, 0, 0)),
        ],
        out_shape=[
            jax.ShapeDtypeStruct((K, T, HD), jnp.float32),
            jax.ShapeDtypeStruct((T, HD), jnp.float32),
            jax.ShapeDtypeStruct((T, HD), jnp.float32),
            jax.ShapeDtypeStruct((T, K * E), jnp.float32),
            jax.ShapeDtypeStruct((NBT, 1, E), jnp.float32),
            jax.ShapeDtypeStruct((NBT, 1, E), jnp.float32),
            jax.ShapeDtypeStruct((NBT, 1, 8), jnp.float32),
        ],
    )(xf, wg, wq_flat, kv_w, kvb2, selmat, red)

    o = pl.pallas_call(
        _attn_kernel,
        grid=(B, K, NQ),
        in_specs=[
            pl.BlockSpec((1, BQ, HD), lambda b, h, i: (h, b * NQ + i, 0)),
            pl.BlockSpec((1, N, HD), lambda b, h, i: (0, b, 0)),
            pl.BlockSpec((1, N, HD), lambda b, h, i: (0, b, 0)),
        ],
        out_specs=pl.BlockSpec((1, BQ, HD), lambda b, h, i: (h, b * NQ + i, 0)),
        out_shape=jax.ShapeDtypeStruct((K, T, HD), jnp.float32),
    )(q, k_.reshape(1, T, HD), v_.reshape(1, T, HD))

    out2d = pl.pallas_call(
        _reduce_kernel,
        grid=(NBT,),
        in_specs=[
            pl.BlockSpec((K, BT, HD), lambda i: (0, i, 0)),
            pl.BlockSpec((BT, K * E), lambda i: (i, 0)),
            pl.BlockSpec((E, EH), lambda i: (0, 0)),
            pl.BlockSpec((HD, EH), lambda i: (0, 0)),
            pl.BlockSpec((EH, DIM), lambda i: (0, 0)),
        ],
        out_specs=pl.BlockSpec((BT, DIM), lambda i: (i, 0)),
        out_shape=jax.ShapeDtypeStruct((T, DIM), jnp.float32),
    )(o, ws, selmat, tilem, wo_flat)

    out = out2d.reshape(B, N, DIM)

    zsum = jnp.sum(zs[:, 0, 0])
    zloss = 0.001 * zsum / T
    freqs = jnp.sum(fr[:, 0, :], axis=0)
    psum = jnp.sum(ps[:, 0, :], axis=0)
    freqs_n = freqs / (jnp.sum(freqs) + 1e-9)
    pm = psum / T
    pm_n = pm / (jnp.sum(pm) + 1e-9)
    switchloss = 0.1 * E * jnp.sum(pm_n * freqs_n)
    aux_loss = zloss + switchloss
    return out, aux_loss


# speedup vs baseline: 3.5626x; 1.1533x over previous
import jax
import jax.numpy as jnp
from jax.experimental import pallas as pl
from jax.experimental.pallas import tpu as pltpu

DIM = 768
E = 16
K = 8
HD = 96
B = 4
N = 2048
T = B * N
BT = 512
BQ = 512
NBT = T // BT
NQ = N // BQ
EH = E * HD
VW = 128


def _route_proj_kernel(x_ref, wg_ref, wq_ref, kvw_ref, kvb_ref,
                       y_ref, k_ref, v_ref, ws_ref, fr_ref, ps_ref, zs_ref):
    x = x_ref[...]
    kv = jnp.dot(x, kvw_ref[...], preferred_element_type=jnp.float32)
    kv = kv + kvb_ref[...]
    k_ref[...] = kv[:, :HD]
    lane = jax.lax.broadcasted_iota(jnp.int32, (BT, VW - HD), 1)
    pad = jnp.where(lane == 0, 1.0, 0.0)
    v_ref[...] = jnp.concatenate([kv[:, HD:], pad], axis=-1)
    logits = jnp.dot(x, wg_ref[...], preferred_element_type=jnp.float32)
    m = jnp.max(logits, axis=-1, keepdims=True)
    ex = jnp.exp(logits - m)
    se = jnp.sum(ex, axis=-1, keepdims=True)
    p = ex / se
    lse = m + jnp.log(se)
    zs_ref[...] = jnp.broadcast_to(jnp.sum(lse * lse), (1, 1, 8))
    eidx = jax.lax.broadcasted_iota(jnp.int32, (BT, E), 1)
    rank = jnp.zeros((BT, E), jnp.int32)
    for j in range(E):
        pj = p[:, j:j + 1]
        rank = rank + jnp.where((pj > p) | ((pj == p) & (j < eidx)), 1, 0)
    sel = rank < K
    self32 = sel.astype(jnp.float32)
    gm = self32 * p
    g = gm / (jnp.sum(gm, axis=-1, keepdims=True) + 1e-6)
    slot = jnp.zeros((BT, E), jnp.int32)
    for j in range(E):
        sj = jnp.where(sel[:, j:j + 1], 1, 0)
        slot = slot + jnp.where(eidx > j, sj, 0)
    y_ref[...] = jnp.dot(x, wq_ref[...], preferred_element_type=jnp.float32)
    for kk in range(K):
        sk = self32 * (slot == kk).astype(jnp.float32)
        ws_ref[kk] = sk * g
    fr_ref[0] = jnp.sum(self32, axis=0, keepdims=True)
    ps_ref[0] = jnp.sum(p, axis=0, keepdims=True)


def _attn_moe_kernel(y_ref, ws_ref, k_ref, v_ref, sel_ref, r_ref, tile_ref,
                     wo_ref, out_ref, z_ref):
    kk = pl.program_id(2)
    selmat = sel_ref[...]
    g = ws_ref[0]
    smask = jnp.where(g > 0, 1.0, 0.0)
    widesel = jnp.dot(smask, selmat, preferred_element_type=jnp.float32)
    q = jnp.dot(widesel * y_ref[...], r_ref[...],
                preferred_element_type=jnp.float32)
    s = jax.lax.dot_general(q, k_ref[0], (((1,), (1,)), ((), ())),
                            preferred_element_type=jnp.float32)
    e = jnp.exp(s)
    oa = jnp.dot(e, v_ref[0], preferred_element_type=jnp.float32)
    o = oa[:, :HD] / oa[:, HD:HD + 1]
    wideg = jnp.dot(g, selmat, preferred_element_type=jnp.float32)
    rep = jnp.dot(o, tile_ref[...], preferred_element_type=jnp.float32)
    contrib = wideg * rep

    @pl.when(kk == 0)
    def _():
        z_ref[...] = contrib

    @pl.when(kk > 0)
    def _():
        z_ref[...] += contrib

    @pl.when(kk == K - 1)
    def _():
        out_ref[...] = jnp.dot(z_ref[...], wo_ref[...],
                               preferred_element_type=jnp.float32)


def kernel(x, w_gate, Wq, kv_w, kv_b, W_out, task_bh):
    xf = x.reshape(T, DIM)
    wg = w_gate[task_bh]
    wq_flat = jnp.transpose(Wq, (1, 0, 2)).reshape(DIM, EH)
    wo_flat = W_out.reshape(EH, DIM)
    kvb2 = kv_b.reshape(1, 2 * HD)
    eye_e = jnp.eye(E, dtype=jnp.float32)
    selmat = jnp.repeat(eye_e, HD, axis=1).reshape(E, EH)
    red = jnp.tile(jnp.eye(HD, dtype=jnp.float32), (E, 1)) * (HD ** -0.5)
    tilem = jnp.tile(jnp.eye(HD, dtype=jnp.float32), (1, E))

    y, k_, v_, ws, fr, ps, zs = pl.pallas_call(
        _route_proj_kernel,
        grid=(NBT,),
        in_specs=[
            pl.BlockSpec((BT, DIM), lambda i: (i, 0)),
            pl.BlockSpec((DIM, E), lambda i: (0, 0)),
            pl.BlockSpec((DIM, EH), lambda i: (0, 0)),
            pl.BlockSpec((DIM, 2 * HD), lambda i: (0, 0)),
            pl.BlockSpec((1, 2 * HD), lambda i: (0, 0)),
        ],
        out_specs=[
            pl.BlockSpec((BT, EH), lambda i: (i, 0)),
            pl.BlockSpec((BT, HD), lambda i: (i, 0)),
            pl.BlockSpec((BT, VW), lambda i: (i, 0)),
            pl.BlockSpec((K, BT, E), lambda i: (0, i, 0)),
            pl.BlockSpec((1, 1, E), lambda i: (i, 0, 0)),
            pl.BlockSpec((1, 1, E), lambda i: (i, 0, 0)),
            pl.BlockSpec((1, 1, 8), lambda i: (i, 0, 0)),
        ],
        out_shape=[
            jax.ShapeDtypeStruct((T, EH), jnp.float32),
            jax.ShapeDtypeStruct((T, HD), jnp.float32),
            jax.ShapeDtypeStruct((T, VW), jnp.float32),
            jax.ShapeDtypeStruct((K, T, E), jnp.float32),
            jax.ShapeDtypeStruct((NBT, 1, E), jnp.float32),
            jax.ShapeDtypeStruct((NBT, 1, E), jnp.float32),
            jax.ShapeDtypeStruct((NBT, 1, 8), jnp.float32),
        ],
    )(xf, wg, wq_flat, kv_w, kvb2)

    out2d = pl.pallas_call(
        _attn_moe_kernel,
        grid=(B, NQ, K),
        in_specs=[
            pl.BlockSpec((BQ, EH), lambda b, i, h: (b * NQ + i, 0)),
            pl.BlockSpec((1, BQ, E), lambda b, i, h: (h, b * NQ + i, 0)),
            pl.BlockSpec((1, N, HD), lambda b, i, h: (0, b, 0)),
            pl.BlockSpec((1, N, VW), lambda b, i, h: (0, b, 0)),
            pl.BlockSpec((E, EH), lambda b, i, h: (0, 0)),
            pl.BlockSpec((EH, HD), lambda b, i, h: (0, 0)),
            pl.BlockSpec((HD, EH), lambda b, i, h: (0, 0)),
            pl.BlockSpec((EH, DIM), lambda b, i, h: (0, 0)),
        ],
        out_specs=pl.BlockSpec((BQ, DIM), lambda b, i, h: (b * NQ + i, 0)),
        out_shape=jax.ShapeDtypeStruct((T, DIM), jnp.float32),
        scratch_shapes=[pltpu.VMEM((BQ, EH), jnp.float32)],
    )(y, ws, k_.reshape(1, T, HD), v_.reshape(1, T, VW), selmat, red, tilem,
      wo_flat)

    out = out2d.reshape(B, N, DIM)

    zsum = jnp.sum(zs[:, 0, 0])
    zloss = 0.001 * zsum / T
    freqs = jnp.sum(fr[:, 0, :], axis=0)
    psum = jnp.sum(ps[:, 0, :], axis=0)
    freqs_n = freqs / (jnp.sum(freqs) + 1e-9)
    pm = psum / T
    pm_n = pm / (jnp.sum(pm) + 1e-9)
    switchloss = 0.1 * E * jnp.sum(pm_n * freqs_n)
    aux_loss = zloss + switchloss
    return out, aux_loss


# speedup vs baseline: 3.7467x; 1.0517x over previous
import jax
import jax.numpy as jnp
from jax.experimental import pallas as pl
from jax.experimental.pallas import tpu as pltpu

DIM = 768
E = 16
K = 8
HD = 96
B = 4
N = 2048
T = B * N
BT = 512
BQ = 512
NBT = T // BT
NQ = N // BQ
EH = E * HD
VW = 128


def _route_proj_kernel(x_ref, wg_ref, wq_ref, kvw_ref, kvb_ref,
                       y_ref, k_ref, v_ref, ws_ref, fr_ref, ps_ref, zs_ref):
    x = x_ref[...]
    kv = jnp.dot(x, kvw_ref[...], preferred_element_type=jnp.float32)
    kv = kv + kvb_ref[...]
    k_ref[...] = kv[:, :HD]
    lane = jax.lax.broadcasted_iota(jnp.int32, (BT, VW - HD), 1)
    pad = jnp.where(lane == 0, 1.0, 0.0)
    v_ref[...] = jnp.concatenate([kv[:, HD:], pad], axis=-1)
    logits = jnp.dot(x, wg_ref[...], preferred_element_type=jnp.float32)
    m = jnp.max(logits, axis=-1, keepdims=True)
    ex = jnp.exp(logits - m)
    se = jnp.sum(ex, axis=-1, keepdims=True)
    p = ex / se
    lse = m + jnp.log(se)
    zs_ref[...] = jnp.broadcast_to(jnp.sum(lse * lse), (1, 1, 8))
    eidx = jax.lax.broadcasted_iota(jnp.int32, (BT, E), 1)
    rank = jnp.zeros((BT, E), jnp.int32)
    for j in range(E):
        pj = p[:, j:j + 1]
        rank = rank + jnp.where((pj > p) | ((pj == p) & (j < eidx)), 1, 0)
    sel = rank < K
    self32 = sel.astype(jnp.float32)
    gm = self32 * p
    g = gm / (jnp.sum(gm, axis=-1, keepdims=True) + 1e-6)
    slot = jnp.zeros((BT, E), jnp.int32)
    for j in range(E):
        sj = jnp.where(sel[:, j:j + 1], 1, 0)
        slot = slot + jnp.where(eidx > j, sj, 0)
    y_ref[...] = jnp.dot(x, wq_ref[...], preferred_element_type=jnp.float32)
    for kk in range(K):
        sk = self32 * (slot == kk).astype(jnp.float32)
        ws_ref[kk] = sk * g
    fr_ref[0] = jnp.sum(self32, axis=0, keepdims=True)
    ps_ref[0] = jnp.sum(p, axis=0, keepdims=True)


def _attn_moe_kernel(y_ref, ws_ref, k_ref, v_ref, sel_ref, r_ref, tile_ref,
                     wo_ref, out_ref, z_ref):
    kk = pl.program_id(2)
    selmat = sel_ref[...]
    g = ws_ref[0]
    smask = jnp.where(g > 0, 1.0, 0.0)
    widesel = jnp.dot(smask, selmat, preferred_element_type=jnp.float32)
    q = jnp.dot(widesel * y_ref[...], r_ref[...],
                preferred_element_type=jnp.float32)
    s = jax.lax.dot_general(q, k_ref[0], (((1,), (1,)), ((), ())),
                            preferred_element_type=jnp.float32)
    e = jnp.exp(s)
    oa = jnp.dot(e, v_ref[0], preferred_element_type=jnp.float32)
    gval = jnp.sum(g, axis=-1, keepdims=True)
    ow = oa[:, :HD] * (gval / oa[:, HD:HD + 1])
    rep = jnp.dot(ow, tile_ref[...], preferred_element_type=jnp.float32)
    contrib = widesel * rep

    @pl.when(kk == 0)
    def _():
        z_ref[...] = contrib

    @pl.when(kk > 0)
    def _():
        z_ref[...] += contrib

    @pl.when(kk == K - 1)
    def _():
        out_ref[...] = jnp.dot(z_ref[...], wo_ref[...],
                               preferred_element_type=jnp.float32)


def kernel(x, w_gate, Wq, kv_w, kv_b, W_out, task_bh):
    xf = x.reshape(T, DIM)
    wg = w_gate[task_bh]
    wq_flat = jnp.transpose(Wq, (1, 0, 2)).reshape(DIM, EH)
    wo_flat = W_out.reshape(EH, DIM)
    kvb2 = kv_b.reshape(1, 2 * HD)
    eye_e = jnp.eye(E, dtype=jnp.float32)
    selmat = jnp.repeat(eye_e, HD, axis=1).reshape(E, EH)
    red = jnp.tile(jnp.eye(HD, dtype=jnp.float32), (E, 1)) * (HD ** -0.5)
    tilem = jnp.tile(jnp.eye(HD, dtype=jnp.float32), (1, E))

    y, k_, v_, ws, fr, ps, zs = pl.pallas_call(
        _route_proj_kernel,
        grid=(NBT,),
        in_specs=[
            pl.BlockSpec((BT, DIM), lambda i: (i, 0)),
            pl.BlockSpec((DIM, E), lambda i: (0, 0)),
            pl.BlockSpec((DIM, EH), lambda i: (0, 0)),
            pl.BlockSpec((DIM, 2 * HD), lambda i: (0, 0)),
            pl.BlockSpec((1, 2 * HD), lambda i: (0, 0)),
        ],
        out_specs=[
            pl.BlockSpec((BT, EH), lambda i: (i, 0)),
            pl.BlockSpec((BT, HD), lambda i: (i, 0)),
            pl.BlockSpec((BT, VW), lambda i: (i, 0)),
            pl.BlockSpec((K, BT, E), lambda i: (0, i, 0)),
            pl.BlockSpec((1, 1, E), lambda i: (i, 0, 0)),
            pl.BlockSpec((1, 1, E), lambda i: (i, 0, 0)),
            pl.BlockSpec((1, 1, 8), lambda i: (i, 0, 0)),
        ],
        out_shape=[
            jax.ShapeDtypeStruct((T, EH), jnp.float32),
            jax.ShapeDtypeStruct((T, HD), jnp.float32),
            jax.ShapeDtypeStruct((T, VW), jnp.float32),
            jax.ShapeDtypeStruct((K, T, E), jnp.float32),
            jax.ShapeDtypeStruct((NBT, 1, E), jnp.float32),
            jax.ShapeDtypeStruct((NBT, 1, E), jnp.float32),
            jax.ShapeDtypeStruct((NBT, 1, 8), jnp.float32),
        ],
    )(xf, wg, wq_flat, kv_w, kvb2)

    out2d = pl.pallas_call(
        _attn_moe_kernel,
        grid=(B, NQ, K),
        in_specs=[
            pl.BlockSpec((BQ, EH), lambda b, i, h: (b * NQ + i, 0)),
            pl.BlockSpec((1, BQ, E), lambda b, i, h: (h, b * NQ + i, 0)),
            pl.BlockSpec((1, N, HD), lambda b, i, h: (0, b, 0)),
            pl.BlockSpec((1, N, VW), lambda b, i, h: (0, b, 0)),
            pl.BlockSpec((E, EH), lambda b, i, h: (0, 0)),
            pl.BlockSpec((EH, HD), lambda b, i, h: (0, 0)),
            pl.BlockSpec((HD, EH), lambda b, i, h: (0, 0)),
            pl.BlockSpec((EH, DIM), lambda b, i, h: (0, 0)),
        ],
        out_specs=pl.BlockSpec((BQ, DIM), lambda b, i, h: (b * NQ + i, 0)),
        out_shape=jax.ShapeDtypeStruct((T, DIM), jnp.float32),
        scratch_shapes=[pltpu.VMEM((BQ, EH), jnp.float32)],
    )(y, ws, k_.reshape(1, T, HD), v_.reshape(1, T, VW), selmat, red, tilem,
      wo_flat)

    out = out2d.reshape(B, N, DIM)

    zsum = jnp.sum(zs[:, 0, 0])
    zloss = 0.001 * zsum / T
    freqs = jnp.sum(fr[:, 0, :], axis=0)
    psum = jnp.sum(ps[:, 0, :], axis=0)
    freqs_n = freqs / (jnp.sum(freqs) + 1e-9)
    pm = psum / T
    pm_n = pm / (jnp.sum(pm) + 1e-9)
    switchloss = 0.1 * E * jnp.sum(pm_n * freqs_n)
    aux_loss = zloss + switchloss
    return out, aux_loss
